# Initial kernel scaffold; baseline (speedup 1.0000x reference)
import functools

import jax
import jax.numpy as jnp
import numpy as np
from jax import lax
from jax.experimental import pallas as pl
from jax.experimental.pallas import tpu as pltpu
from jax.experimental.pallas import tpu_sc as plsc

N = 10000
E = 320000
NFEAT = 128
NHID = 64
NCLASS = 40
NLAYERS = 8
ALPHA = 0.1
LAMDA = 0.5

NC = 2
NS = 16
NW = NC * NS
L = 16
EPW = E // NW
CHUNK = 80
NCHUNK = EPW // CHUNK
RPW = N // NS
ZROWS = 125


def _spmm_sc(h, src, dst, w):
    mesh = plsc.VectorSubcoreMesh(core_axis_name="c", subcore_axis_name="s")

    @functools.partial(
        pl.kernel,
        mesh=mesh,
        out_type=jax.ShapeDtypeStruct((NC, N, NHID), jnp.float32),
        scratch_types=[
            pltpu.VMEM((CHUNK,), jnp.int32),
            pltpu.VMEM((CHUNK,), jnp.int32),
            pltpu.VMEM((CHUNK,), jnp.float32),
            pltpu.VMEM((CHUNK, NHID), jnp.float32),
            pltpu.VMEM((ZROWS, NHID), jnp.float32),
            pltpu.VMEM_SHARED((N, NHID), jnp.float32),
        ],
    )
    def k(h_hbm, src_hbm, dst_hbm, w_hbm, out_hbm, src_v, dst_v, w_v, rows_v,
          z_v, agg_sh):
        cid = lax.axis_index("c")
        sid = lax.axis_index("s")
        wid = cid * NS + sid

        @pl.loop(0, ZROWS)
        def _(r):
            for c in range(NHID // L):
                z_v[r, pl.ds(c * L, L)] = jnp.zeros((L,), jnp.float32)

        @pl.loop(0, RPW // ZROWS)
        def _(j):
            pltpu.sync_copy(z_v, agg_sh.at[pl.ds(sid * RPW + j * ZROWS, ZROWS)])

        plsc.subcore_barrier()

        base_w = wid * EPW

        @pl.loop(0, NCHUNK)
        def _(ck):
            base = base_w + ck * CHUNK
            pltpu.sync_copy(src_hbm.at[pl.ds(base, CHUNK)], src_v)
            pltpu.sync_copy(dst_hbm.at[pl.ds(base, CHUNK)], dst_v)
            pltpu.sync_copy(w_hbm.at[pl.ds(base, CHUNK)], w_v)
            pltpu.sync_copy(h_hbm.at[src_v], rows_v)

            @pl.loop(0, CHUNK)
            def _(r):
                idx = jnp.broadcast_to(r, (L,)).astype(jnp.int32)
                wv = plsc.load_gather(w_v, [idx])
                for c in range(NHID // L):
                    sl = pl.ds(c * L, L)
                    rows_v[r, sl] = rows_v[r, sl] * wv

            pltpu.sync_copy(rows_v, agg_sh.at[dst_v], add=True)

        plsc.subcore_barrier()

        @pl.loop(0, RPW // ZROWS)
        def _(j):
            row0 = sid * RPW + j * ZROWS
            pltpu.sync_copy(agg_sh.at[pl.ds(row0, ZROWS)],
                            out_hbm.at[cid, pl.ds(row0, ZROWS)])

    return k(h, src, dst, w)


def _fc1(x, W1, b1):
    def body(x_ref, w_ref, b_ref, o_ref):
        o_ref[...] = jax.nn.relu(
            jnp.dot(x_ref[...], w_ref[...], preferred_element_type=jnp.float32)
            + b_ref[...])

    return pl.pallas_call(
        body,
        out_shape=jax.ShapeDtypeStruct((N, NHID), jnp.float32),
    )(x, W1, b1.reshape(1, NHID))


def _layer_update(p, h0, M):
    def body(p_ref, h0_ref, m_ref, o_ref):
        s = (1.0 - ALPHA) * (p_ref[0] + p_ref[1]) + ALPHA * h0_ref[...]
        o_ref[...] = jax.nn.relu(
            jnp.dot(s, m_ref[...], preferred_element_type=jnp.float32))

    return pl.pallas_call(
        body,
        out_shape=jax.ShapeDtypeStruct((N, NHID), jnp.float32),
    )(p, h0, M)


def _fc_out(h, W2, b2):
    def body(h_ref, w_ref, b_ref, o_ref):
        logits = (jnp.dot(h_ref[...], w_ref[...],
                          preferred_element_type=jnp.float32) + b_ref[...])
        m = jnp.max(logits, axis=1, keepdims=True)
        lse = jnp.log(jnp.sum(jnp.exp(logits - m), axis=1, keepdims=True)) + m
        o_ref[...] = logits - lse

    return pl.pallas_call(
        body,
        out_shape=jax.ShapeDtypeStruct((N, NCLASS), jnp.float32),
    )(h, W2, b2.reshape(1, NCLASS))


def kernel(x, edge_index, edge_weight, W1, b1, Wc, W2, b2):
    src = edge_index[0]
    dst = edge_index[1]
    thetas = np.log(LAMDA / (np.arange(1, NLAYERS + 1)) + 1.0).astype(np.float32)
    eye = jnp.eye(NHID, dtype=jnp.float32)
    M = (jnp.asarray(thetas)[:, None, None] * Wc
         + (1.0 - jnp.asarray(thetas))[:, None, None] * eye[None])

    h = _fc1(x, W1, b1)
    h0 = h
    for i in range(NLAYERS):
        p = _spmm_sc(h, src, dst, edge_weight)
        h = _layer_update(p, h0, M[i])
    return _fc_out(h, W2, b2)


# speedup vs baseline: 3.5233x; 3.5233x over previous
import dataclasses
import functools

import jax
import jax.numpy as jnp
import numpy as np
from jax import lax
from jax.experimental import pallas as pl
from jax.experimental.pallas import tpu as pltpu
from jax.experimental.pallas import tpu_sc as plsc

N = 10000
E = 320000
NFEAT = 128
NHID = 64
NCLASS = 40
NLAYERS = 8
ALPHA = 0.1
LAMDA = 0.5

NC = 2
NS = 16
NW = NC * NS
L = 16
EPW = E // NW
CHUNK = 80
NCHUNK = EPW // CHUNK
BLK = 200
NBLK = N // BLK


def _spmm_sc(h, src, dst, w):
    mesh = plsc.VectorSubcoreMesh(core_axis_name="c", subcore_axis_name="s")
    cp = pltpu.CompilerParams()
    for f, v in (("needs_layout_passes", False),
                 ("use_tc_tiling_on_sc", False)):
        if f in pltpu.CompilerParams.__dataclass_fields__:
            cp = dataclasses.replace(cp, **{f: v})

    @functools.partial(
        pl.kernel,
        mesh=mesh,
        compiler_params=cp,
        out_type=jax.ShapeDtypeStruct((NC, N, NHID), jnp.float32),
        scratch_types=[
            pltpu.VMEM((CHUNK,), jnp.int32),
            pltpu.VMEM((CHUNK,), jnp.int32),
            pltpu.VMEM((CHUNK,), jnp.float32),
            pltpu.VMEM((CHUNK, NHID), jnp.float32),
            pltpu.VMEM((BLK, NHID), jnp.float32),
            pltpu.VMEM_SHARED((N, NHID), jnp.float32),
        ],
    )
    def k(h_hbm, src_hbm, dst_hbm, w_hbm, out_hbm, src_v, dst_v, w_v, rows_v,
          z_v, agg_sh):
        cid = lax.axis_index("c")
        sid = lax.axis_index("s")
        wid = cid * NS + sid

        @pl.loop(0, BLK)
        def _(r):
            for c in range(NHID // L):
                z_v[r, pl.ds(c * L, L)] = jnp.zeros((L,), jnp.float32)

        for j in range((NBLK + NS - 1) // NS):
            b = j * NS + sid

            @pl.when(b < NBLK)
            def _():
                pltpu.sync_copy(z_v, agg_sh.at[pl.ds(b * BLK, BLK)])

        plsc.subcore_barrier()

        base_w = wid * EPW

        @pl.loop(0, NCHUNK)
        def _(ck):
            base = base_w + ck * CHUNK
            pltpu.sync_copy(src_hbm.at[pl.ds(base, CHUNK)], src_v)
            pltpu.sync_copy(dst_hbm.at[pl.ds(base, CHUNK)], dst_v)
            pltpu.sync_copy(w_hbm.at[pl.ds(base, CHUNK)], w_v)
            pltpu.sync_copy(h_hbm.at[src_v], rows_v)

            @pl.loop(0, CHUNK)
            def _(r):
                idx = jnp.broadcast_to(r, (L,)).astype(jnp.int32)
                wv = plsc.load_gather(w_v, [idx])
                for c in range(NHID // L):
                    sl = pl.ds(c * L, L)
                    rows_v[r, sl] = rows_v[r, sl] * wv

            pltpu.sync_copy(rows_v, agg_sh.at[dst_v], add=True)

        plsc.subcore_barrier()

        for j in range((NBLK + NS - 1) // NS):
            b = j * NS + sid

            @pl.when(b < NBLK)
            def _():
                pltpu.sync_copy(agg_sh.at[pl.ds(b * BLK, BLK)],
                                out_hbm.at[cid, pl.ds(b * BLK, BLK)])

    return k(h, src, dst, w)


def _fc1(x, W1, b1):
    def body(x_ref, w_ref, b_ref, o_ref):
        o_ref[...] = jax.nn.relu(
            jnp.dot(x_ref[...], w_ref[...], preferred_element_type=jnp.float32)
            + b_ref[...])

    return pl.pallas_call(
        body,
        out_shape=jax.ShapeDtypeStruct((N, NHID), jnp.float32),
    )(x, W1, b1.reshape(1, NHID))


def _layer_update(p, h0, M):
    def body(p_ref, h0_ref, m_ref, o_ref):
        s = (1.0 - ALPHA) * (p_ref[0] + p_ref[1]) + ALPHA * h0_ref[...]
        o_ref[...] = jax.nn.relu(
            jnp.dot(s, m_ref[...], preferred_element_type=jnp.float32))

    return pl.pallas_call(
        body,
        out_shape=jax.ShapeDtypeStruct((N, NHID), jnp.float32),
    )(p, h0, M)


def _fc_out(h, W2, b2):
    def body(h_ref, w_ref, b_ref, o_ref):
        logits = (jnp.dot(h_ref[...], w_ref[...],
                          preferred_element_type=jnp.float32) + b_ref[...])
        m = jnp.max(logits, axis=1, keepdims=True)
        lse = jnp.log(jnp.sum(jnp.exp(logits - m), axis=1, keepdims=True)) + m
        o_ref[...] = logits - lse

    return pl.pallas_call(
        body,
        out_shape=jax.ShapeDtypeStruct((N, NCLASS), jnp.float32),
    )(h, W2, b2.reshape(1, NCLASS))


def kernel(x, edge_index, edge_weight, W1, b1, Wc, W2, b2):
    src = edge_index[0]
    dst = edge_index[1]
    thetas = np.log(LAMDA / (np.arange(1, NLAYERS + 1)) + 1.0).astype(np.float32)
    eye = jnp.eye(NHID, dtype=jnp.float32)
    M = (jnp.asarray(thetas)[:, None, None] * Wc
         + (1.0 - jnp.asarray(thetas))[:, None, None] * eye[None])

    h = _fc1(x, W1, b1)
    h0 = h
    for i in range(NLAYERS):
        p = _spmm_sc(h, src, dst, edge_weight)
        h = _layer_update(p, h0, M[i])
    return _fc_out(h, W2, b2)


# speedup vs baseline: 12.2817x; 3.4858x over previous
import dataclasses
import functools

import jax
import jax.numpy as jnp
import numpy as np
from jax import lax
from jax.experimental import pallas as pl
from jax.experimental.pallas import tpu as pltpu
from jax.experimental.pallas import tpu_sc as plsc

N = 10000
E = 320000
NFEAT = 128
NHID = 64
NCLASS = 40
NLAYERS = 8
ALPHA = 0.1
LAMDA = 0.5

NC = 2
NS = 16
NW = NC * NS
L = 16
EPW = E // NW
CHUNK = 40
NCHUNK = EPW // CHUNK
BLK = 200
NBLK = N // BLK
NBUF = 5


def _spmm_sc(h, src3, dst3, w2):
    mesh = plsc.VectorSubcoreMesh(core_axis_name="c", subcore_axis_name="s")
    cp = pltpu.CompilerParams()
    for f, v in (("needs_layout_passes", False),
                 ("use_tc_tiling_on_sc", False)):
        if f in pltpu.CompilerParams.__dataclass_fields__:
            cp = dataclasses.replace(cp, **{f: v})

    @functools.partial(
        pl.kernel,
        mesh=mesh,
        compiler_params=cp,
        out_type=jax.ShapeDtypeStruct((NC, N, NHID), jnp.float32),
        scratch_types=[
            pltpu.VMEM((NCHUNK, CHUNK), jnp.int32),
            pltpu.VMEM((NCHUNK, CHUNK), jnp.int32),
            pltpu.VMEM((EPW,), jnp.float32),
            pltpu.VMEM((NBUF, CHUNK, NHID), jnp.float32),
            pltpu.VMEM((NBUF, CHUNK, NHID), jnp.float32),
            pltpu.VMEM_SHARED((N, NHID), jnp.float32),
            pltpu.SemaphoreType.DMA((NBUF,)),
            pltpu.SemaphoreType.DMA((NBUF,)),
        ],
    )
    def k(h_hbm, src_hbm, dst_hbm, w_hbm, out_hbm, src_v, dst_v, w_v,
          rows_g, rows_s, agg_sh, sem_g, sem_s):
        cid = lax.axis_index("c")
        sid = lax.axis_index("s")
        wid = cid * NS + sid

        @pl.loop(0, CHUNK)
        def _(r):
            for c in range(NHID // L):
                rows_s[0, r, pl.ds(c * L, L)] = jnp.zeros((L,), jnp.float32)

        for j in range((NCHUNK + NS - 1) // NS):
            b = j * NS + sid

            @pl.when(b < NCHUNK)
            def _():
                pltpu.sync_copy(rows_s.at[0],
                                agg_sh.at[pl.ds(b * CHUNK, CHUNK)])

        plsc.subcore_barrier()

        pltpu.sync_copy(src_hbm.at[wid], src_v)
        pltpu.sync_copy(dst_hbm.at[wid], dst_v)
        pltpu.sync_copy(w_hbm.at[wid], w_v)

        for b in range(NBUF):
            pltpu.async_copy(h_hbm.at[src_v.at[b]], rows_g.at[b], sem_g.at[b])

        @pl.loop(0, NCHUNK, step=NBUF)
        def _(k0):
            for b in range(NBUF):
                ck = k0 + b
                pltpu.make_async_copy(h_hbm.at[src_v.at[ck]], rows_g.at[b],
                                      sem_g.at[b]).wait()

                @pl.when(k0 > 0)
                def _():
                    pltpu.make_async_copy(
                        rows_s.at[b], agg_sh.at[dst_v.at[ck - NBUF]],
                        sem_s.at[b]).wait()

                @pl.loop(0, CHUNK)
                def _(r):
                    idx = jnp.broadcast_to(ck * CHUNK + r, (L,)).astype(
                        jnp.int32)
                    wv = plsc.load_gather(w_v, [idx])
                    for c in range(NHID // L):
                        sl = pl.ds(c * L, L)
                        rows_s[b, r, sl] = rows_g[b, r, sl] * wv

                pltpu.async_copy(rows_s.at[b], agg_sh.at[dst_v.at[ck]],
                                 sem_s.at[b], add=True)

                @pl.when(ck + NBUF < NCHUNK)
                def _():
                    pltpu.async_copy(h_hbm.at[src_v.at[ck + NBUF]],
                                     rows_g.at[b], sem_g.at[b])

        for b in range(NBUF):
            pltpu.make_async_copy(rows_s.at[b],
                                  agg_sh.at[dst_v.at[NCHUNK - NBUF + b]],
                                  sem_s.at[b]).wait()

        plsc.subcore_barrier()

        for j in range((NBLK + NS - 1) // NS):
            b = j * NS + sid

            @pl.when(b < NBLK)
            def _():
                pltpu.sync_copy(agg_sh.at[pl.ds(b * BLK, BLK)],
                                out_hbm.at[cid, pl.ds(b * BLK, BLK)])

    return k(h, src3, dst3, w2)


def _fc1(x, W1, b1):
    def body(x_ref, w_ref, b_ref, o_ref):
        o_ref[...] = jax.nn.relu(
            jnp.dot(x_ref[...], w_ref[...], preferred_element_type=jnp.float32)
            + b_ref[...])

    return pl.pallas_call(
        body,
        out_shape=jax.ShapeDtypeStruct((N, NHID), jnp.float32),
    )(x, W1, b1.reshape(1, NHID))


def _layer_update(p, h0, M):
    def body(p_ref, h0_ref, m_ref, o_ref):
        s = (1.0 - ALPHA) * (p_ref[0] + p_ref[1]) + ALPHA * h0_ref[...]
        o_ref[...] = jax.nn.relu(
            jnp.dot(s, m_ref[...], preferred_element_type=jnp.float32))

    return pl.pallas_call(
        body,
        out_shape=jax.ShapeDtypeStruct((N, NHID), jnp.float32),
    )(p, h0, M)


def _fc_out(h, W2, b2):
    def body(h_ref, w_ref, b_ref, o_ref):
        logits = (jnp.dot(h_ref[...], w_ref[...],
                          preferred_element_type=jnp.float32) + b_ref[...])
        m = jnp.max(logits, axis=1, keepdims=True)
        lse = jnp.log(jnp.sum(jnp.exp(logits - m), axis=1, keepdims=True)) + m
        o_ref[...] = logits - lse

    return pl.pallas_call(
        body,
        out_shape=jax.ShapeDtypeStruct((N, NCLASS), jnp.float32),
    )(h, W2, b2.reshape(1, NCLASS))


def kernel(x, edge_index, edge_weight, W1, b1, Wc, W2, b2):
    src3 = edge_index[0].reshape(NW, NCHUNK, CHUNK)
    dst3 = edge_index[1].reshape(NW, NCHUNK, CHUNK)
    w2 = edge_weight.reshape(NW, EPW)
    thetas = np.log(LAMDA / (np.arange(1, NLAYERS + 1)) + 1.0).astype(np.float32)
    eye = jnp.eye(NHID, dtype=jnp.float32)
    M = (jnp.asarray(thetas)[:, None, None] * Wc
         + (1.0 - jnp.asarray(thetas))[:, None, None] * eye[None])

    h = _fc1(x, W1, b1)
    h0 = h
    for i in range(NLAYERS):
        p = _spmm_sc(h, src3, dst3, w2)
        h = _layer_update(p, h0, M[i])
    return _fc_out(h, W2, b2)


# speedup vs baseline: 15.1240x; 1.2314x over previous
import dataclasses
import functools

import jax
import jax.numpy as jnp
import numpy as np
from jax import lax
from jax.experimental import pallas as pl
from jax.experimental.pallas import tpu as pltpu
from jax.experimental.pallas import tpu_sc as plsc

N = 10000
E = 320000
NFEAT = 128
NHID = 64
NCLASS = 40
NLAYERS = 8
ALPHA = 0.1
LAMDA = 0.5

NC = 2
NS = 16
NW = NC * NS
L = 16
EPW = E // NW
CHUNK = 40
NCHUNK = EPW // CHUNK
BLK = 200
NBLK = N // BLK
NBUF = 5


def _spmm_sc(h, src3, dst3, w2):
    mesh = plsc.VectorSubcoreMesh(core_axis_name="c", subcore_axis_name="s")
    cp = pltpu.CompilerParams()
    for f, v in (("needs_layout_passes", False),
                 ("use_tc_tiling_on_sc", False)):
        if f in pltpu.CompilerParams.__dataclass_fields__:
            cp = dataclasses.replace(cp, **{f: v})

    @functools.partial(
        pl.kernel,
        mesh=mesh,
        compiler_params=cp,
        out_type=jax.ShapeDtypeStruct((NC, N, NHID), jnp.float32),
        scratch_types=[
            pltpu.VMEM((NCHUNK, CHUNK), jnp.int32),
            pltpu.VMEM((NCHUNK, CHUNK), jnp.int32),
            pltpu.VMEM((EPW,), jnp.float32),
            pltpu.VMEM((NBUF, CHUNK, NHID), jnp.float32),
            pltpu.VMEM((NBUF, CHUNK, NHID), jnp.float32),
            pltpu.VMEM_SHARED((N, NHID), jnp.float32),
            pltpu.SemaphoreType.DMA((NBUF,)),
            pltpu.SemaphoreType.DMA((NBUF,)),
        ],
    )
    def k(h_hbm, src_hbm, dst_hbm, w_hbm, out_hbm, src_v, dst_v, w_v,
          rows_g, rows_s, agg_sh, sem_g, sem_s):
        cid = lax.axis_index("c")
        sid = lax.axis_index("s")
        wid = cid * NS + sid

        @pl.loop(0, CHUNK)
        def _(r):
            for c in range(NHID // L):
                rows_s[0, r, pl.ds(c * L, L)] = jnp.zeros((L,), jnp.float32)

        for j in range((NCHUNK + NS - 1) // NS):
            b = j * NS + sid

            @pl.when(b < NCHUNK)
            def _():
                pltpu.sync_copy(rows_s.at[0],
                                agg_sh.at[pl.ds(b * CHUNK, CHUNK)])

        plsc.subcore_barrier()

        pltpu.sync_copy(src_hbm.at[wid], src_v)
        pltpu.sync_copy(dst_hbm.at[wid], dst_v)
        pltpu.sync_copy(w_hbm.at[wid], w_v)

        for b in range(NBUF):
            pltpu.async_copy(h_hbm.at[src_v.at[b]], rows_g.at[b], sem_g.at[b])

        @pl.loop(0, NCHUNK, step=NBUF)
        def _(k0):
            for b in range(NBUF):
                ck = k0 + b
                pltpu.make_async_copy(h_hbm.at[src_v.at[ck]], rows_g.at[b],
                                      sem_g.at[b]).wait()

                @pl.when(k0 > 0)
                def _():
                    pltpu.make_async_copy(
                        rows_s.at[b], agg_sh.at[dst_v.at[ck - NBUF]],
                        sem_s.at[b]).wait()

                @plsc.parallel_loop(0, CHUNK, unroll=8)
                def _(r):
                    idx = jnp.broadcast_to(ck * CHUNK + r, (L,)).astype(
                        jnp.int32)
                    wv = plsc.load_gather(w_v, [idx])
                    for c in range(NHID // L):
                        sl = pl.ds(c * L, L)
                        rows_s[b, r, sl] = rows_g[b, r, sl] * wv

                pltpu.async_copy(rows_s.at[b], agg_sh.at[dst_v.at[ck]],
                                 sem_s.at[b], add=True)

                @pl.when(ck + NBUF < NCHUNK)
                def _():
                    pltpu.async_copy(h_hbm.at[src_v.at[ck + NBUF]],
                                     rows_g.at[b], sem_g.at[b])

        for b in range(NBUF):
            pltpu.make_async_copy(rows_s.at[b],
                                  agg_sh.at[dst_v.at[NCHUNK - NBUF + b]],
                                  sem_s.at[b]).wait()

        plsc.subcore_barrier()

        for j in range((NBLK + NS - 1) // NS):
            b = j * NS + sid

            @pl.when(b < NBLK)
            def _():
                pltpu.sync_copy(agg_sh.at[pl.ds(b * BLK, BLK)],
                                out_hbm.at[cid, pl.ds(b * BLK, BLK)])

    return k(h, src3, dst3, w2)


def _fc1(x, W1, b1):
    def body(x_ref, w_ref, b_ref, o_ref):
        o_ref[...] = jax.nn.relu(
            jnp.dot(x_ref[...], w_ref[...], preferred_element_type=jnp.float32)
            + b_ref[...])

    return pl.pallas_call(
        body,
        out_shape=jax.ShapeDtypeStruct((N, NHID), jnp.float32),
    )(x, W1, b1.reshape(1, NHID))


def _layer_update(p, h0, M):
    def body(p_ref, h0_ref, m_ref, o_ref):
        s = (1.0 - ALPHA) * (p_ref[0] + p_ref[1]) + ALPHA * h0_ref[...]
        o_ref[...] = jax.nn.relu(
            jnp.dot(s, m_ref[...], preferred_element_type=jnp.float32))

    return pl.pallas_call(
        body,
        out_shape=jax.ShapeDtypeStruct((N, NHID), jnp.float32),
    )(p, h0, M)


def _fc_out(h, W2, b2):
    def body(h_ref, w_ref, b_ref, o_ref):
        logits = (jnp.dot(h_ref[...], w_ref[...],
                          preferred_element_type=jnp.float32) + b_ref[...])
        m = jnp.max(logits, axis=1, keepdims=True)
        lse = jnp.log(jnp.sum(jnp.exp(logits - m), axis=1, keepdims=True)) + m
        o_ref[...] = logits - lse

    return pl.pallas_call(
        body,
        out_shape=jax.ShapeDtypeStruct((N, NCLASS), jnp.float32),
    )(h, W2, b2.reshape(1, NCLASS))


def kernel(x, edge_index, edge_weight, W1, b1, Wc, W2, b2):
    src3 = edge_index[0].reshape(NW, NCHUNK, CHUNK)
    dst3 = edge_index[1].reshape(NW, NCHUNK, CHUNK)
    w2 = edge_weight.reshape(NW, EPW)
    thetas = np.log(LAMDA / (np.arange(1, NLAYERS + 1)) + 1.0).astype(np.float32)
    eye = jnp.eye(NHID, dtype=jnp.float32)
    M = (jnp.asarray(thetas)[:, None, None] * Wc
         + (1.0 - jnp.asarray(thetas))[:, None, None] * eye[None])

    h = _fc1(x, W1, b1)
    h0 = h
    for i in range(NLAYERS):
        p = _spmm_sc(h, src3, dst3, w2)
        h = _layer_update(p, h0, M[i])
    return _fc_out(h, W2, b2)
